# baseline (speedup 1.0000x reference)
import functools

import jax
import jax.numpy as jnp
from jax import lax
from jax.experimental import pallas as pl
from jax.experimental.pallas import tpu as pltpu
from jax.experimental.pallas import tpu_sc as plsc

BATCH = 16384
DIM = 32
NSLOT = 8
NC, NS = 2, 16
NW = NC * NS
BPW = BATCH // NW
CHUNK = 128
NCHUNK = BPW // CHUNK
OUT_D = NSLOT * DIM
NBUF = 2

_mesh = plsc.VectorSubcoreMesh(core_axis_name="c", subcore_axis_name="s")


@functools.partial(
    pl.kernel,
    out_type=jax.ShapeDtypeStruct((BATCH, OUT_D), jnp.float32),
    mesh=_mesh,
    scratch_types=[
        pltpu.VMEM((NSLOT, NCHUNK, CHUNK), jnp.int32),
        pltpu.VMEM((NBUF, NSLOT, CHUNK, DIM), jnp.float32),
        pltpu.SemaphoreType.DMA,
        pltpu.SemaphoreType.DMA,
        pltpu.SemaphoreType.DMA,
        pltpu.SemaphoreType.DMA,
    ],
    compiler_params=pltpu.CompilerParams(use_tc_tiling_on_sc=False),
)
def _gather_concat(idx_hbm, paper_hbm, pfield_hbm, author_hbm, year_hbm,
                   oa_hbm, out_hbm, idx_v, slot_v, gsem0, gsem1, wsem0,
                   wsem1):
    wid = lax.axis_index("s") * NC + lax.axis_index("c")
    base = wid * BPW
    tables = (paper_hbm, pfield_hbm, pfield_hbm, author_hbm, author_hbm,
              author_hbm, year_hbm, oa_hbm)
    gsems = (gsem0, gsem1)
    wsems = (wsem0, wsem1)
    pltpu.sync_copy(idx_hbm.at[wid], idx_v)

    def issue_gathers(c):
        buf = c % NBUF
        return [
            pltpu.async_copy(tab.at[idx_v.at[s, c]], slot_v.at[buf, s],
                             gsems[buf])
            for s, tab in enumerate(tables)
        ]

    def issue_writes(c):
        buf = c % NBUF
        rb = base + c * CHUNK
        return [
            pltpu.async_copy(
                slot_v.at[buf, s],
                out_hbm.at[pl.ds(rb, CHUNK), pl.ds(s * DIM, DIM)],
                wsems[buf])
            for s in range(NSLOT)
        ]

    gathers = [None] * NBUF
    writes = [None] * NBUF
    gathers[0] = issue_gathers(0)
    for c in range(NCHUNK):
        buf = c % NBUF
        if c + 1 < NCHUNK:
            obuf = (c + 1) % NBUF
            if writes[obuf] is not None:
                for w in writes[obuf]:
                    w.wait()
            gathers[obuf] = issue_gathers(c + 1)
        for g in gathers[buf]:
            g.wait()
        writes[buf] = issue_writes(c)
    for ws in writes:
        if ws is not None:
            for w in ws:
                w.wait()


def kernel(paperId, fieldsOfStudy_0, fieldsOfStudy_1, authors_0, authors_1,
           authors_2, year, isOpenAccess, paper_table, pfield_table,
           author_table, year_table, oa_table):
    idx = jnp.stack([paperId, fieldsOfStudy_0, fieldsOfStudy_1, authors_0,
                     authors_1, authors_2, year, isOpenAccess])
    idx = (idx.astype(jnp.int32)
              .reshape(NSLOT, NW, NCHUNK, CHUNK)
              .transpose(1, 0, 2, 3))
    return _gather_concat(idx, paper_table, pfield_table, author_table,
                          year_table, oa_table)
